# baseline (speedup 1.0000x reference)
import functools

import jax
import jax.numpy as jnp
from jax import lax
from jax.experimental import pallas as pl
from jax.experimental.pallas import tpu as pltpu
from jax.experimental.pallas import tpu_sc as plsc

B = 16384
L = 20
EMB = 32
TYPE_V = 101
AUD_V = 21
GENRE_V = 10000
NC = 2
NS = 16
NW = NC * NS
BPW = B // NW

_mesh = plsc.VectorSubcoreMesh(
    core_axis_name="c", subcore_axis_name="s", num_cores=NC, num_subcores=NS
)

_f32 = jnp.float32


@functools.partial(
    pl.kernel,
    out_type=jax.ShapeDtypeStruct((B, 3 * EMB), _f32),
    mesh=_mesh,
    compiler_params=pltpu.CompilerParams(use_tc_tiling_on_sc=False),
    scratch_types=[
        pltpu.VMEM((L + 2, BPW), jnp.int32),
        pltpu.VMEM((BPW, EMB), _f32),
        pltpu.VMEM((TYPE_V, EMB), _f32),
        pltpu.VMEM((AUD_V, EMB), _f32),
        pltpu.VMEM((BPW,), _f32),
        pltpu.VMEM((BPW,), _f32),
        pltpu.VMEM((EMB,), _f32),
        pltpu.VMEM((BPW, 3 * EMB), _f32),
        pltpu.SemaphoreType.DMA,
    ],
)
def _sc_embed(
    idall_hbm, tabs_hbm, out_hbm,
    gid_v, acc_v, ttab_v, atab_v, n0_v, rec_v, row0_v,
    out_v, gsem,
):
    wid = lax.axis_index("c") * NS + lax.axis_index("s")
    base = wid * BPW

    pltpu.sync_copy(idall_hbm.at[:, pl.ds(base, BPW)], gid_v)

    pltpu.async_copy(tabs_hbm.at[gid_v.at[0]], acc_v, gsem)

    pltpu.sync_copy(tabs_hbm.at[pl.ds(GENRE_V, TYPE_V), :], ttab_v)
    pltpu.sync_copy(tabs_hbm.at[pl.ds(GENRE_V + TYPE_V, AUD_V), :], atab_v)
    pltpu.sync_copy(tabs_hbm.at[0], row0_v)

    pltpu.make_async_copy(tabs_hbm.at[gid_v.at[0]], acc_v, gsem).wait()

    def fire(l, _):
        pltpu.async_copy(tabs_hbm.at[gid_v.at[l]], acc_v, gsem, add=True)
        return 0

    lax.fori_loop(1, L, fire, 0)

    def count_body(g, _):
        off = g * 16
        acc = jnp.zeros((16,), _f32)
        for l in range(L):
            ids = gid_v[l, pl.ds(off, 16)]
            acc = acc + jnp.where(ids == 0, 1.0, 0.0).astype(_f32)
        n0_v[pl.ds(off, 16)] = acc
        cnt = jnp.float32(L) - acc
        rec_v[pl.ds(off, 16)] = jnp.where(
            cnt > 0.5, jnp.float32(1.0) / cnt, jnp.float32(0.0)
        )
        tidg = gid_v[L, pl.ds(off, 16)]
        aidg = gid_v[L + 1, pl.ds(off, 16)]
        for j in range(16):
            r = off + j
            t = tidg[j]
            a = aidg[j]
            out_v[r, pl.ds(EMB, 16)] = ttab_v[t, pl.ds(0, 16)]
            out_v[r, pl.ds(EMB + 16, 16)] = ttab_v[t, pl.ds(16, 16)]
            out_v[r, pl.ds(2 * EMB, 16)] = atab_v[a, pl.ds(0, 16)]
            out_v[r, pl.ds(2 * EMB + 16, 16)] = atab_v[a, pl.ds(16, 16)]
        return 0

    lax.fori_loop(0, BPW // 16, count_body, 0)

    def drain(l, _):
        pltpu.make_async_copy(tabs_hbm.at[gid_v.at[0]], acc_v, gsem).wait()
        return 0

    lax.fori_loop(1, L, drain, 0)

    r0a = row0_v[pl.ds(0, 16)]
    r0b = row0_v[pl.ds(16, 16)]

    def norm(g, _):
        off = g * 16
        n0g = n0_v[pl.ds(off, 16)]
        recg = rec_v[pl.ds(off, 16)]
        for j in range(16):
            r = off + j
            n0 = n0g[j]
            rec = recg[j]
            v0 = acc_v[r, pl.ds(0, 16)]
            v1 = acc_v[r, pl.ds(16, 16)]
            out_v[r, pl.ds(0, 16)] = (v0 - n0 * r0a) * rec
            out_v[r, pl.ds(16, 16)] = (v1 - n0 * r0b) * rec
        return 0

    lax.fori_loop(0, BPW // 16, norm, 0)

    pltpu.sync_copy(out_v, out_hbm.at[pl.ds(base, BPW), :])


def kernel(genre_ids, type_ids, audience_ids, genre_table, type_table,
           audience_table):
    gids = genre_ids.astype(jnp.int32)
    tids = type_ids.astype(jnp.int32)
    aids = audience_ids.astype(jnp.int32)
    idall = jnp.concatenate([gids.T, tids[None, :], aids[None, :]], axis=0)
    tabs = jnp.concatenate([genre_table, type_table, audience_table],
                           axis=0)
    return _sc_embed(idall, tabs)


# speedup vs baseline: 1.0954x; 1.0954x over previous
import functools

import jax
import jax.numpy as jnp
from jax import lax
from jax.experimental import pallas as pl
from jax.experimental.pallas import tpu as pltpu
from jax.experimental.pallas import tpu_sc as plsc

B = 16384
L = 20
EMB = 32
TYPE_V = 101
AUD_V = 21
NC = 2
NS = 16
NW = NC * NS
BPW = B // NW

_mesh = plsc.VectorSubcoreMesh(
    core_axis_name="c", subcore_axis_name="s", num_cores=NC, num_subcores=NS
)

_f32 = jnp.float32


@functools.partial(
    pl.kernel,
    out_type=jax.ShapeDtypeStruct((B * 3 * EMB,), _f32),
    mesh=_mesh,
    compiler_params=pltpu.CompilerParams(use_tc_tiling_on_sc=False),
    scratch_types=[
        pltpu.VMEM((L, 1, BPW), jnp.int32),
        pltpu.VMEM((BPW,), jnp.int32),
        pltpu.VMEM((BPW,), jnp.int32),
        pltpu.VMEM((BPW, EMB), _f32),
        pltpu.VMEM((TYPE_V, EMB), _f32),
        pltpu.VMEM((AUD_V, EMB), _f32),
        pltpu.VMEM((BPW,), _f32),
        pltpu.VMEM((BPW,), _f32),
        pltpu.VMEM((EMB,), _f32),
        pltpu.VMEM((BPW * 3 * EMB,), _f32),
        pltpu.SemaphoreType.DMA,
    ],
)
def _sc_embed(
    gidx_hbm, tid_hbm, aid_hbm, gtab, ttab, atab, out_hbm,
    gid_v, tid_v, aid_v, acc_v, ttab_v, atab_v, n0_v, rec_v, row0_v,
    out_v, gsem,
):
    wid = lax.axis_index("c") * NS + lax.axis_index("s")
    base = wid * BPW

    pltpu.sync_copy(gidx_hbm.at[:, pl.ds(wid, 1), :], gid_v)

    pltpu.async_copy(gtab.at[gid_v.at[0, 0]], acc_v, gsem)

    pltpu.sync_copy(tid_hbm.at[pl.ds(base, BPW)], tid_v)
    pltpu.sync_copy(aid_hbm.at[pl.ds(base, BPW)], aid_v)
    pltpu.sync_copy(ttab, ttab_v)
    pltpu.sync_copy(atab, atab_v)
    pltpu.sync_copy(gtab.at[0], row0_v)

    pltpu.make_async_copy(gtab.at[gid_v.at[0, 0]], acc_v, gsem).wait()

    def fire(l, _):
        pltpu.async_copy(gtab.at[gid_v.at[l, 0]], acc_v, gsem, add=True)
        return 0

    lax.fori_loop(1, L, fire, 0)

    def count_body(g, _):
        off = g * 16
        acc = jnp.zeros((16,), _f32)
        for l in range(L):
            ids = gid_v[l, 0, pl.ds(off, 16)]
            acc = acc + jnp.where(ids == 0, 1.0, 0.0).astype(_f32)
        n0_v[pl.ds(off, 16)] = acc
        cnt = jnp.float32(L) - acc
        rec_v[pl.ds(off, 16)] = jnp.where(
            cnt > 0.5, jnp.float32(1.0) / cnt, jnp.float32(0.0)
        )
        tidg = tid_v[pl.ds(off, 16)]
        aidg = aid_v[pl.ds(off, 16)]
        for j in range(16):
            r96 = (off + j) * 3 * EMB
            t = tidg[j]
            a = aidg[j]
            out_v[pl.ds(r96 + EMB, 16)] = ttab_v[t, pl.ds(0, 16)]
            out_v[pl.ds(r96 + EMB + 16, 16)] = ttab_v[t, pl.ds(16, 16)]
            out_v[pl.ds(r96 + 2 * EMB, 16)] = atab_v[a, pl.ds(0, 16)]
            out_v[pl.ds(r96 + 2 * EMB + 16, 16)] = atab_v[a, pl.ds(16, 16)]
        return 0

    lax.fori_loop(0, BPW // 16, count_body, 0)

    def drain(l, _):
        pltpu.make_async_copy(gtab.at[gid_v.at[0, 0]], acc_v, gsem).wait()
        return 0

    lax.fori_loop(1, L, drain, 0)

    r0a = row0_v[pl.ds(0, 16)]
    r0b = row0_v[pl.ds(16, 16)]

    def norm(g, _):
        off = g * 16
        n0g = n0_v[pl.ds(off, 16)]
        recg = rec_v[pl.ds(off, 16)]
        for j in range(16):
            r = off + j
            r96 = r * 3 * EMB
            n0 = n0g[j]
            rec = recg[j]
            v0 = acc_v[r, pl.ds(0, 16)]
            v1 = acc_v[r, pl.ds(16, 16)]
            out_v[pl.ds(r96, 16)] = (v0 - n0 * r0a) * rec
            out_v[pl.ds(r96 + 16, 16)] = (v1 - n0 * r0b) * rec
        return 0

    lax.fori_loop(0, BPW // 16, norm, 0)

    pltpu.sync_copy(out_v, out_hbm.at[pl.ds(base * 3 * EMB, BPW * 3 * EMB)])


def kernel(genre_ids, type_ids, audience_ids, genre_table, type_table,
           audience_table):
    gids = genre_ids.astype(jnp.int32)
    tids = type_ids.astype(jnp.int32)
    aids = audience_ids.astype(jnp.int32)
    gidx = gids.T.reshape(L, NW, BPW)
    out = _sc_embed(gidx, tids, aids, genre_table, type_table,
                    audience_table)
    return out.reshape(B, 3 * EMB)


# speedup vs baseline: 1.0971x; 1.0016x over previous
import functools

import jax
import jax.numpy as jnp
from jax import lax
from jax.experimental import pallas as pl
from jax.experimental.pallas import tpu as pltpu
from jax.experimental.pallas import tpu_sc as plsc

B = 16384
L = 20
EMB = 32
TYPE_V = 101
AUD_V = 21
NC = 2
NS = 16
NW = NC * NS
BPW = B // NW

_mesh = plsc.VectorSubcoreMesh(
    core_axis_name="c", subcore_axis_name="s", num_cores=NC, num_subcores=NS
)

_f32 = jnp.float32


@functools.partial(
    pl.kernel,
    out_type=jax.ShapeDtypeStruct((B, 3 * EMB), _f32),
    mesh=_mesh,
    compiler_params=pltpu.CompilerParams(use_tc_tiling_on_sc=False),
    scratch_types=[
        pltpu.VMEM((L, 1, BPW), jnp.int32),
        pltpu.VMEM((BPW,), jnp.int32),
        pltpu.VMEM((BPW,), jnp.int32),
        pltpu.VMEM((BPW, EMB), _f32),
        pltpu.VMEM((TYPE_V, EMB), _f32),
        pltpu.VMEM((AUD_V, EMB), _f32),
        pltpu.VMEM((BPW,), _f32),
        pltpu.VMEM((BPW,), _f32),
        pltpu.VMEM((EMB,), _f32),
        pltpu.VMEM((BPW, 3 * EMB), _f32),
        pltpu.SemaphoreType.DMA,
    ],
)
def _sc_embed(
    gidx_hbm, tid_hbm, aid_hbm, gtab, ttab, atab, out_hbm,
    gid_v, tid_v, aid_v, acc_v, ttab_v, atab_v, n0_v, rec_v, row0_v,
    out_v, gsem,
):
    wid = lax.axis_index("c") * NS + lax.axis_index("s")
    base = wid * BPW

    pltpu.sync_copy(gidx_hbm.at[:, pl.ds(wid, 1), :], gid_v)

    pltpu.async_copy(gtab.at[gid_v.at[0, 0]], acc_v, gsem)

    pltpu.sync_copy(tid_hbm.at[pl.ds(base, BPW)], tid_v)
    pltpu.sync_copy(aid_hbm.at[pl.ds(base, BPW)], aid_v)
    pltpu.sync_copy(ttab, ttab_v)
    pltpu.sync_copy(atab, atab_v)
    pltpu.sync_copy(gtab.at[0], row0_v)

    pltpu.make_async_copy(gtab.at[gid_v.at[0, 0]], acc_v, gsem).wait()

    def fire(l, _):
        pltpu.async_copy(gtab.at[gid_v.at[l, 0]], acc_v, gsem, add=True)
        return 0

    lax.fori_loop(1, L, fire, 0)

    def count_body(g, _):
        off = g * 16
        acc = jnp.zeros((16,), _f32)
        for l in range(L):
            ids = gid_v[l, 0, pl.ds(off, 16)]
            acc = acc + jnp.where(ids == 0, 1.0, 0.0).astype(_f32)
        n0_v[pl.ds(off, 16)] = acc
        cnt = jnp.float32(L) - acc
        rec_v[pl.ds(off, 16)] = jnp.where(
            cnt > 0.5, jnp.float32(1.0) / cnt, jnp.float32(0.0)
        )
        tidg = tid_v[pl.ds(off, 16)]
        aidg = aid_v[pl.ds(off, 16)]
        for j in range(16):
            r = off + j
            t = tidg[j]
            a = aidg[j]
            out_v[r, pl.ds(EMB, 16)] = ttab_v[t, pl.ds(0, 16)]
            out_v[r, pl.ds(EMB + 16, 16)] = ttab_v[t, pl.ds(16, 16)]
            out_v[r, pl.ds(2 * EMB, 16)] = atab_v[a, pl.ds(0, 16)]
            out_v[r, pl.ds(2 * EMB + 16, 16)] = atab_v[a, pl.ds(16, 16)]
        return 0

    lax.fori_loop(0, BPW // 16, count_body, 0)

    def drain(l, _):
        pltpu.make_async_copy(gtab.at[gid_v.at[0, 0]], acc_v, gsem).wait()
        return 0

    lax.fori_loop(1, L, drain, 0)

    r0a = row0_v[pl.ds(0, 16)]
    r0b = row0_v[pl.ds(16, 16)]

    def norm(g, _):
        off = g * 16
        n0g = n0_v[pl.ds(off, 16)]
        recg = rec_v[pl.ds(off, 16)]
        for j in range(16):
            r = off + j
            n0 = n0g[j]
            rec = recg[j]
            v0 = acc_v[r, pl.ds(0, 16)]
            v1 = acc_v[r, pl.ds(16, 16)]
            out_v[r, pl.ds(0, 16)] = (v0 - n0 * r0a) * rec
            out_v[r, pl.ds(16, 16)] = (v1 - n0 * r0b) * rec
        return 0

    lax.fori_loop(0, BPW // 16, norm, 0)

    pltpu.sync_copy(out_v, out_hbm.at[pl.ds(base, BPW), :])


def kernel(genre_ids, type_ids, audience_ids, genre_table, type_table,
           audience_table):
    gids = genre_ids.astype(jnp.int32)
    tids = type_ids.astype(jnp.int32)
    aids = audience_ids.astype(jnp.int32)
    gidx = gids.T.reshape(L, NW, BPW)
    return _sc_embed(gidx, tids, aids, genre_table, type_table,
                     audience_table)
